# baseline (speedup 1.0000x reference)
import functools

import jax
import jax.numpy as jnp
from jax import lax
from jax.experimental import pallas as pl
from jax.experimental.pallas import tpu as pltpu
from jax.experimental.pallas import tpu_sc as plsc

_VOCAB = 100000
_EMBED = 128
_BATCH = 4096
_CTX = 20

_GATHER_CHUNK = 128


def _sc_gather(x_flat, emb_table):
    n = x_flat.shape[0]
    info = plsc.get_sparse_core_info()
    n_workers = info.num_cores * info.num_subcores
    per_w = n // n_workers
    n_chunks = per_w // _GATHER_CHUNK
    mesh = plsc.VectorSubcoreMesh(core_axis_name="c", subcore_axis_name="s")

    @functools.partial(
        pl.kernel,
        out_type=jax.ShapeDtypeStruct((n, _EMBED), jnp.float32),
        mesh=mesh,
        scratch_types=[
            pltpu.VMEM((_GATHER_CHUNK,), jnp.int32),
            pltpu.VMEM((_GATHER_CHUNK, _EMBED), jnp.float32),
            pltpu.SemaphoreType.DMA,
        ],
    )
    def gather_kernel(idx_hbm, table_hbm, out_hbm, idx_v, rows_v, sem):
        wid = lax.axis_index("s") * info.num_cores + lax.axis_index("c")
        base = wid * per_w

        @pl.loop(0, n_chunks)
        def _chunk(ci):
            off = base + ci * _GATHER_CHUNK
            pltpu.sync_copy(idx_hbm.at[pl.ds(off, _GATHER_CHUNK)], idx_v)
            pltpu.async_copy(table_hbm.at[idx_v], rows_v, sem).wait()
            pltpu.sync_copy(rows_v, out_hbm.at[pl.ds(off, _GATHER_CHUNK)])

    return gather_kernel(x_flat, emb_table)


def _mean_pool(gathered):
    bb = 512

    def mean_kernel(rows_ref, out_ref):
        r = rows_ref[...].reshape(bb, _CTX, _EMBED)
        out_ref[...] = jnp.sum(r, axis=1) * (1.0 / _CTX)

    return pl.pallas_call(
        mean_kernel,
        grid=(_BATCH // bb,),
        in_specs=[pl.BlockSpec((bb * _CTX, _EMBED), lambda i: (i, 0))],
        out_specs=pl.BlockSpec((bb, _EMBED), lambda i: (i, 0)),
        out_shape=jax.ShapeDtypeStruct((_BATCH, _EMBED), jnp.float32),
    )(gathered)


def _projection(h, W, b2d):
    vb = 1024
    nv = pl.cdiv(_VOCAB, vb)

    def mm_kernel(h_ref, w_ref, b_ref, out_ref):
        acc = lax.dot_general(
            h_ref[...], w_ref[...],
            (((1,), (1,)), ((), ())),
            preferred_element_type=jnp.float32,
        )
        out_ref[...] = acc + b_ref[...]

    return pl.pallas_call(
        mm_kernel,
        grid=(nv,),
        in_specs=[
            pl.BlockSpec((_BATCH, _EMBED), lambda j: (0, 0)),
            pl.BlockSpec((vb, _EMBED), lambda j: (j, 0)),
            pl.BlockSpec((1, vb), lambda j: (0, j)),
        ],
        out_specs=pl.BlockSpec((_BATCH, vb), lambda j: (0, j)),
        out_shape=jax.ShapeDtypeStruct((_BATCH, _VOCAB), jnp.float32),
    )(h, W, b2d)


def kernel(x, emb_table, W, b):
    x_flat = x.reshape(-1).astype(jnp.int32)
    gathered = _sc_gather(x_flat, emb_table)
    h = _mean_pool(gathered)
    return _projection(h, W, b.reshape(1, _VOCAB))
